# baseline (speedup 1.0000x reference)
import functools

import jax
import jax.numpy as jnp
from jax import lax
from jax.experimental import pallas as pl
from jax.experimental.pallas import tpu as pltpu
from jax.experimental.pallas import tpu_sc as plsc

D = 32
NC, NS = 2, 16
NW = NC * NS
BLK = 256


@functools.partial(jax.jit, static_argnums=(2, 3))
def _gather(x_flat, weight, hist, batch):
    blocks_per_h = batch // BLK
    n_items = hist * blocks_per_h
    items_per_w = n_items // NW
    mesh = plsc.VectorSubcoreMesh(core_axis_name="c", subcore_axis_name="s")

    @functools.partial(
        pl.kernel,
        out_type=jax.ShapeDtypeStruct((hist, D // 8, batch // 128, 8, 128),
                                      jnp.float32),
        mesh=mesh,
        scratch_types=[
            pltpu.VMEM((2, BLK), jnp.int32),
            pltpu.VMEM((2, BLK, D), jnp.float32),
            pltpu.VMEM((2, D // 8, BLK // 128, 8, 128), jnp.float32),
            pltpu.SemaphoreType.DMA,
            pltpu.SemaphoreType.DMA,
            pltpu.SemaphoreType.DMA,
            pltpu.SemaphoreType.DMA,
            pltpu.SemaphoreType.DMA,
            pltpu.SemaphoreType.DMA,
        ],
        compiler_params=pltpu.CompilerParams(
            use_tc_tiling_on_sc=False, needs_layout_passes=False),
    )
    def body(x_hbm, table_hbm, out_hbm, idx_v, rows_v, tr_v,
             sem_g0, sem_g1, sem_i0, sem_i1, sem_s0, sem_s1):
        wid = lax.axis_index("s") * NC + lax.axis_index("c")
        base_item = wid * items_per_w
        lanes = lax.iota(jnp.int32, 16)
        sem_g = (sem_g0, sem_g1)
        sem_i = (sem_i0, sem_i1)
        sem_s = (sem_s0, sem_s1)

        def idx_src(item):
            return x_hbm.at[pl.ds(item * BLK, BLK)]

        def gather_cp(item, b):
            return pltpu.make_async_copy(
                table_hbm.at[idx_v.at[b]], rows_v.at[b], sem_g[b])

        def store_cp(item, b):
            h = item // blocks_per_h
            jb0 = (item % blocks_per_h) * (BLK // 128)
            return pltpu.make_async_copy(
                tr_v.at[b],
                out_hbm.at[h, :, pl.ds(jb0, BLK // 128)], sem_s[b])

        def transpose_block(b):
            @plsc.parallel_loop(0, BLK // 16, unroll=1)
            def tr_body(j):
                jj = j // 8
                l0 = (j % 8) * 16
                row_idx = j * 16 + lanes
                for d in range(D):
                    val = plsc.load_gather(
                        rows_v.at[b], [row_idx, jnp.full((16,), d, jnp.int32)])
                    tr_v[b, d // 8, jj, d % 8, pl.ds(l0, 16)] = val

        pltpu.sync_copy(idx_src(base_item), idx_v.at[0])
        pltpu.async_copy(idx_src(base_item + 1), idx_v.at[1], sem_i[1])
        gather_cp(base_item, 0).start()

        def iter_body(g, carry):
            for b in range(2):
                t = g * 2 + b
                item = base_item + t
                nb = 1 - b
                gather_cp(item, b).wait()

                @pl.when(t + 2 < items_per_w)
                def _():
                    pltpu.async_copy(idx_src(item + 2), idx_v.at[b], sem_i[b])

                @pl.when(t + 1 < items_per_w)
                def _():
                    pltpu.make_async_copy(
                        idx_src(item + 1), idx_v.at[nb], sem_i[nb]).wait()
                    gather_cp(item + 1, nb).start()

                @pl.when(t >= 2)
                def _():
                    store_cp(item - 2, b).wait()

                transpose_block(b)
                store_cp(item, b).start()
            return carry

        lax.fori_loop(0, items_per_w // 2, iter_body, 0)
        store_cp(base_item + items_per_w - 2, items_per_w % 2).wait()
        store_cp(base_item + items_per_w - 1, (items_per_w - 1) % 2).wait()

    return body(x_flat, weight)


def kernel(x, weight):
    batch, hist = x.shape
    xt_flat = jnp.transpose(x).reshape(hist * batch)
    y5 = _gather(xt_flat, weight, hist, batch)
    return jnp.transpose(y5, (2, 4, 0, 1, 3)).reshape(batch, hist, D)


# speedup vs baseline: 1.0165x; 1.0165x over previous
import functools

import jax
import jax.numpy as jnp
from jax import lax
from jax.experimental import pallas as pl
from jax.experimental.pallas import tpu as pltpu
from jax.experimental.pallas import tpu_sc as plsc

D = 32
NC, NS = 2, 16
NW = NC * NS
BLK = 512


@functools.partial(jax.jit, static_argnums=(2, 3))
def _gather(x_flat, weight, hist, batch):
    blocks_per_h = batch // BLK
    n_items = hist * blocks_per_h
    items_per_w = n_items // NW
    mesh = plsc.VectorSubcoreMesh(core_axis_name="c", subcore_axis_name="s")

    @functools.partial(
        pl.kernel,
        out_type=jax.ShapeDtypeStruct((hist, D // 8, batch // 128, 8, 128),
                                      jnp.float32),
        mesh=mesh,
        scratch_types=[
            pltpu.VMEM((2, BLK), jnp.int32),
            pltpu.VMEM((2, BLK, D), jnp.float32),
            pltpu.VMEM((2, D // 8, BLK // 128, 8, 128), jnp.float32),
            pltpu.SemaphoreType.DMA,
            pltpu.SemaphoreType.DMA,
            pltpu.SemaphoreType.DMA,
            pltpu.SemaphoreType.DMA,
            pltpu.SemaphoreType.DMA,
            pltpu.SemaphoreType.DMA,
        ],
        compiler_params=pltpu.CompilerParams(
            use_tc_tiling_on_sc=False, needs_layout_passes=False),
    )
    def body(x_hbm, table_hbm, out_hbm, idx_v, rows_v, tr_v,
             sem_g0, sem_g1, sem_i0, sem_i1, sem_s0, sem_s1):
        wid = lax.axis_index("s") * NC + lax.axis_index("c")
        base_item = wid * items_per_w
        lanes = lax.iota(jnp.int32, 16)
        sem_g = (sem_g0, sem_g1)
        sem_i = (sem_i0, sem_i1)
        sem_s = (sem_s0, sem_s1)

        def idx_src(item):
            return x_hbm.at[pl.ds(item * BLK, BLK)]

        def gather_cp(item, b):
            return pltpu.make_async_copy(
                table_hbm.at[idx_v.at[b]], rows_v.at[b], sem_g[b])

        def store_cp(item, b):
            h = item // blocks_per_h
            jb0 = (item % blocks_per_h) * (BLK // 128)
            return pltpu.make_async_copy(
                tr_v.at[b],
                out_hbm.at[h, :, pl.ds(jb0, BLK // 128)], sem_s[b])

        def transpose_block(b):
            @plsc.parallel_loop(0, BLK // 16, unroll=1)
            def tr_body(j):
                jj = j // 8
                l0 = (j % 8) * 16
                row_idx = j * 16 + lanes
                for d in range(D):
                    val = plsc.load_gather(
                        rows_v.at[b], [row_idx, jnp.full((16,), d, jnp.int32)])
                    tr_v[b, d // 8, jj, d % 8, pl.ds(l0, 16)] = val

        pltpu.sync_copy(idx_src(base_item), idx_v.at[0])
        pltpu.async_copy(idx_src(base_item + 1), idx_v.at[1], sem_i[1])
        gather_cp(base_item, 0).start()

        def iter_body(g, carry):
            for b in range(2):
                t = g * 2 + b
                item = base_item + t
                nb = 1 - b
                gather_cp(item, b).wait()

                @pl.when(t + 2 < items_per_w)
                def _():
                    pltpu.async_copy(idx_src(item + 2), idx_v.at[b], sem_i[b])

                @pl.when(t + 1 < items_per_w)
                def _():
                    pltpu.make_async_copy(
                        idx_src(item + 1), idx_v.at[nb], sem_i[nb]).wait()
                    gather_cp(item + 1, nb).start()

                @pl.when(t >= 2)
                def _():
                    store_cp(item - 2, b).wait()

                transpose_block(b)
                store_cp(item, b).start()
            return carry

        lax.fori_loop(0, items_per_w // 2, iter_body, 0)
        store_cp(base_item + items_per_w - 2, items_per_w % 2).wait()
        store_cp(base_item + items_per_w - 1, (items_per_w - 1) % 2).wait()

    return body(x_flat, weight)


def kernel(x, weight):
    batch, hist = x.shape
    xt_flat = jnp.transpose(x).reshape(hist * batch)
    y5 = _gather(xt_flat, weight, hist, batch)
    return jnp.transpose(y5, (2, 4, 0, 1, 3)).reshape(batch, hist, D)
